# baseline (speedup 1.0000x reference)
import jax
import jax.numpy as jnp
from jax.experimental import pallas as pl
from jax.experimental.pallas import tpu as pltpu


def kernel(x, table):
    B, S = x.shape
    M, D = table.shape
    TILE = 8192
    N = S // TILE

    def body(tab_hbm, out_hbm, buf, in_sem, out_sem):
        def in_copy(i, p):
            return pltpu.make_async_copy(
                tab_hbm.at[pl.ds(i * TILE, TILE), :], buf.at[p], in_sem.at[p]
            )

        def out_copy(i, p, b):
            return pltpu.make_async_copy(
                buf.at[p], out_hbm.at[b, pl.ds(i * TILE, TILE), :], out_sem.at[p]
            )

        in_copy(0, 0).start()
        for i in range(N):
            p = i % 2
            if i + 1 < N:
                if i >= 1:
                    for b in range(B):
                        out_copy(i - 1, p ^ 1, b).wait()
                in_copy(i + 1, p ^ 1).start()
            in_copy(i, p).wait()
            for b in range(B):
                out_copy(i, p, b).start()
        for i in range(max(0, N - 2), N):
            for b in range(B):
                out_copy(i, i % 2, b).wait()

    out = pl.pallas_call(
        body,
        in_specs=[pl.BlockSpec(memory_space=pltpu.MemorySpace.HBM)],
        out_specs=pl.BlockSpec(memory_space=pltpu.MemorySpace.HBM),
        out_shape=jax.ShapeDtypeStruct((B, S, D), table.dtype),
        scratch_shapes=[
            pltpu.VMEM((min(2, N), TILE, D), table.dtype),
            pltpu.SemaphoreType.DMA((2,)),
            pltpu.SemaphoreType.DMA((2,)),
        ],
    )(table)
    return out


# speedup vs baseline: 1.0482x; 1.0482x over previous
import jax
import jax.numpy as jnp
from jax.experimental import pallas as pl
from jax.experimental.pallas import tpu as pltpu


def kernel(x, table):
    B, S = x.shape
    M, D = table.shape
    sizes = [1024, 1024, 2048, 4096]
    assert sum(sizes) == S
    offs = [sum(sizes[:i]) for i in range(len(sizes))]
    N = len(sizes)
    BUF = max(sizes)

    def body(tab_hbm, out_hbm, buf, in_sem, out_sem):
        def in_copy(i, p):
            return pltpu.make_async_copy(
                tab_hbm.at[pl.ds(offs[i], sizes[i]), :],
                buf.at[p, pl.ds(0, sizes[i]), :],
                in_sem.at[p],
            )

        def out_copy(i, p, b):
            return pltpu.make_async_copy(
                buf.at[p, pl.ds(0, sizes[i]), :],
                out_hbm.at[b, pl.ds(offs[i], sizes[i]), :],
                out_sem.at[p],
            )

        in_copy(0, 0).start()
        for i in range(N):
            p = i % 2
            if i + 1 < N:
                if i >= 1:
                    for b in range(B):
                        out_copy(i - 1, p ^ 1, b).wait()
                in_copy(i + 1, p ^ 1).start()
            in_copy(i, p).wait()
            for b in range(B):
                out_copy(i, p, b).start()
        for i in range(max(0, N - 2), N):
            for b in range(B):
                out_copy(i, i % 2, b).wait()

    out = pl.pallas_call(
        body,
        in_specs=[pl.BlockSpec(memory_space=pltpu.MemorySpace.HBM)],
        out_specs=pl.BlockSpec(memory_space=pltpu.MemorySpace.HBM),
        out_shape=jax.ShapeDtypeStruct((B, S, D), table.dtype),
        scratch_shapes=[
            pltpu.VMEM((2, BUF, D), table.dtype),
            pltpu.SemaphoreType.DMA((2,)),
            pltpu.SemaphoreType.DMA((2,)),
        ],
    )(table)
    return out


# speedup vs baseline: 1.0965x; 1.0461x over previous
import jax
import jax.numpy as jnp
from jax.experimental import pallas as pl
from jax.experimental.pallas import tpu as pltpu


def kernel(x, table):
    B, S = x.shape
    M, D = table.shape
    sizes = [4096, 4096]
    assert sum(sizes) == S
    offs = [sum(sizes[:i]) for i in range(len(sizes))]
    N = len(sizes)
    BUF = max(sizes)

    def body(tab_hbm, out_hbm, buf, in_sem, out_sem):
        def in_copy(i, p):
            return pltpu.make_async_copy(
                tab_hbm.at[pl.ds(offs[i], sizes[i]), :],
                buf.at[p, pl.ds(0, sizes[i]), :],
                in_sem.at[p],
            )

        def out_copy(i, p, b):
            return pltpu.make_async_copy(
                buf.at[p, pl.ds(0, sizes[i]), :],
                out_hbm.at[b, pl.ds(offs[i], sizes[i]), :],
                out_sem.at[p],
            )

        in_copy(0, 0).start()
        for i in range(N):
            p = i % 2
            if i + 1 < N:
                if i >= 1:
                    for b in range(B):
                        out_copy(i - 1, p ^ 1, b).wait()
                in_copy(i + 1, p ^ 1).start()
            in_copy(i, p).wait()
            for b in range(B):
                out_copy(i, p, b).start()
        for i in range(max(0, N - 2), N):
            for b in range(B):
                out_copy(i, i % 2, b).wait()

    out = pl.pallas_call(
        body,
        in_specs=[pl.BlockSpec(memory_space=pltpu.MemorySpace.HBM)],
        out_specs=pl.BlockSpec(memory_space=pltpu.MemorySpace.HBM),
        out_shape=jax.ShapeDtypeStruct((B, S, D), table.dtype),
        scratch_shapes=[
            pltpu.VMEM((2, BUF, D), table.dtype),
            pltpu.SemaphoreType.DMA((2,)),
            pltpu.SemaphoreType.DMA((2,)),
        ],
    )(table)
    return out


# speedup vs baseline: 1.0982x; 1.0015x over previous
import jax
import jax.numpy as jnp
from jax.experimental import pallas as pl
from jax.experimental.pallas import tpu as pltpu


def kernel(x, table):
    B, S = x.shape
    M, D = table.shape
    sizes = [4096, 4096]
    assert sum(sizes) == S
    offs = [sum(sizes[:i]) for i in range(len(sizes))]
    N = len(sizes)
    BUF = max(sizes)

    def body(tab_hbm, out_hbm, buf, in_sem, out_sem):
        def in_copy(i, p):
            return pltpu.make_async_copy(
                tab_hbm.at[pl.ds(offs[i], sizes[i]), :],
                buf.at[p, pl.ds(0, sizes[i]), :],
                in_sem.at[p],
            )

        def out_copies(i, p, b):
            h = sizes[i] // 2
            return [
                pltpu.make_async_copy(
                    buf.at[p, pl.ds(k * h, h), :],
                    out_hbm.at[b, pl.ds(offs[i] + k * h, h), :],
                    out_sem.at[p],
                )
                for k in range(2)
            ]

        in_copy(0, 0).start()
        for i in range(N):
            p = i % 2
            if i + 1 < N:
                if i >= 1:
                    for b in range(B):
                        for c in out_copies(i - 1, p ^ 1, b):
                            c.wait()
                in_copy(i + 1, p ^ 1).start()
            in_copy(i, p).wait()
            for b in range(B):
                for c in out_copies(i, p, b):
                    c.start()
        for i in range(max(0, N - 2), N):
            for b in range(B):
                for c in out_copies(i, i % 2, b):
                    c.wait()

    out = pl.pallas_call(
        body,
        in_specs=[pl.BlockSpec(memory_space=pltpu.MemorySpace.HBM)],
        out_specs=pl.BlockSpec(memory_space=pltpu.MemorySpace.HBM),
        out_shape=jax.ShapeDtypeStruct((B, S, D), table.dtype),
        scratch_shapes=[
            pltpu.VMEM((2, BUF, D), table.dtype),
            pltpu.SemaphoreType.DMA((2,)),
            pltpu.SemaphoreType.DMA((2,)),
        ],
    )(table)
    return out
